# baseline (speedup 1.0000x reference)
import jax
import jax.numpy as jnp
from jax import lax
from jax.experimental import pallas as pl
from jax.experimental.pallas import tpu as pltpu
from jax.experimental.pallas import tpu_sc as plsc

RANK = 128
HALF = RANK // 2
MARGIN = 1.0
NC = 2
NS = 16
NW = NC * NS
L = 16


def _sc_transe(b_per_w):
    def body(idxh_hbm, idxr_hbm, idxt_hbm, idxs_hbm, e0, e1, e2,
             pred_out, head_out, rel_out, tail_out, ts_out,
             ih_v, it_v, ir_v, is_v,
             head_v, rel_v, tail_v, ts_v, sos_v, pred_v,
             gsem, rsem, wsem):
        wid = lax.axis_index("s") * NC + lax.axis_index("c")
        base = wid * b_per_w

        pltpu.sync_copy(idxh_hbm.at[pl.ds(base, b_per_w)], ih_v)
        pltpu.sync_copy(idxt_hbm.at[pl.ds(base, b_per_w)], it_v)
        pltpu.sync_copy(idxr_hbm.at[pl.ds(base, b_per_w)], ir_v)
        pltpu.sync_copy(idxs_hbm.at[pl.ds(base, b_per_w)], is_v)

        g1 = pltpu.async_copy(e0.at[ih_v], head_v, gsem)
        g3 = pltpu.async_copy(e0.at[it_v], tail_v, gsem)

        def rel_dma_body(g, carry):
            rbase = g * L
            vr = ir_v[pl.ds(rbase, L)]
            for j in range(L):
                pltpu.async_copy(e1.at[vr[j]], rel_v.at[rbase + j], rsem)
            return carry

        def ts_dma_body(g, carry):
            rbase = g * L
            vs = is_v[pl.ds(rbase, L)]
            for j in range(L):
                pltpu.async_copy(e2.at[vs[j]], ts_v.at[rbase + j], rsem)
            return carry

        lax.fori_loop(0, b_per_w // L, rel_dma_body, 0)
        lax.fori_loop(0, b_per_w // L, ts_dma_body, 0)

        g1.wait()
        g3.wait()
        pltpu.make_async_copy(e1.at[pl.ds(0, b_per_w)], rel_v, rsem).wait()
        pltpu.make_async_copy(e2.at[pl.ds(0, b_per_w)], ts_v, rsem).wait()

        w1 = pltpu.async_copy(head_v, head_out.at[pl.ds(base, b_per_w)], wsem)
        w2 = pltpu.async_copy(rel_v, rel_out.at[pl.ds(base, b_per_w)], wsem)
        w3 = pltpu.async_copy(tail_v, tail_out.at[pl.ds(base, b_per_w)], wsem)
        w4 = pltpu.async_copy(ts_v, ts_out.at[pl.ds(base, b_per_w)], wsem)

        lanes = lax.iota(jnp.int32, L)

        def row_body(r, carry):
            acc = jnp.zeros((L,), jnp.float32)
            for k in range(RANK // L):
                h = head_v[r, pl.ds(k * L, L)]
                t = tail_v[r, pl.ds(k * L, L)]
                if k < HALF // L:
                    rt = rel_v[r, pl.ds(k * L, L)]
                else:
                    rt = ts_v[r, pl.ds(k * L - HALF, L)]
                d = h + rt - t
                acc = acc + d * d
            plsc.store_scatter(sos_v, [lanes, jnp.broadcast_to(r, (L,))], acc)
            return carry

        lax.fori_loop(0, b_per_w, row_body, 0)

        for g in range(b_per_w // L):
            s = sos_v[0, pl.ds(g * L, L)]
            for l in range(1, L):
                s = s + sos_v[l, pl.ds(g * L, L)]
            sc = jnp.maximum(s, 1e-30)
            i = lax.bitcast_convert_type(sc, jnp.int32)
            i = jnp.int32(0x5F3759DF) - lax.shift_right_arithmetic(i, 1)
            y = lax.bitcast_convert_type(i, jnp.float32)
            for _ in range(4):
                y = y * (1.5 - 0.5 * sc * y * y)
            pred_v[pl.ds(g * L, L)] = MARGIN - s * y

        pltpu.sync_copy(pred_v, pred_out.at[pl.ds(base, b_per_w)])
        w1.wait()
        w2.wait()
        w3.wait()
        w4.wait()

    return body


def kernel(x_data, E0, E1, E2, bh, bt):
    del bh, bt
    B = x_data.shape[0]
    b_per_w = B // NW
    idx_h = x_data[:, 0]
    idx_r = x_data[:, 1]
    idx_t = x_data[:, 2]
    idx_s = x_data[:, 3]

    mesh = plsc.VectorSubcoreMesh(core_axis_name="c", subcore_axis_name="s")
    out_type = (
        jax.ShapeDtypeStruct((B,), jnp.float32),
        jax.ShapeDtypeStruct((B, RANK), jnp.float32),
        jax.ShapeDtypeStruct((B, HALF), jnp.float32),
        jax.ShapeDtypeStruct((B, RANK), jnp.float32),
        jax.ShapeDtypeStruct((B, HALF), jnp.float32),
    )
    scratch = [
        pltpu.VMEM((b_per_w,), jnp.int32),
        pltpu.VMEM((b_per_w,), jnp.int32),
        pltpu.VMEM((b_per_w,), jnp.int32),
        pltpu.VMEM((b_per_w,), jnp.int32),
        pltpu.VMEM((b_per_w, RANK), jnp.float32),
        pltpu.VMEM((b_per_w, HALF), jnp.float32),
        pltpu.VMEM((b_per_w, RANK), jnp.float32),
        pltpu.VMEM((b_per_w, HALF), jnp.float32),
        pltpu.VMEM((L, b_per_w), jnp.float32),
        pltpu.VMEM((b_per_w,), jnp.float32),
        pltpu.SemaphoreType.DMA,
        pltpu.SemaphoreType.DMA,
        pltpu.SemaphoreType.DMA,
    ]
    fn = pl.kernel(_sc_transe(b_per_w), out_type=out_type, mesh=mesh,
                   scratch_types=scratch,
                   compiler_params=pltpu.CompilerParams(
                       needs_layout_passes=False))
    preds, head_e, rel_e, tail_e, ts_e = fn(
        idx_h, idx_r, idx_t, idx_s, E0, E1, E2)
    return (preds, (head_e, rel_e, tail_e, ts_e))
